# baseline (speedup 1.0000x reference)
import functools

import jax
import jax.numpy as jnp
from jax import lax
from jax.experimental import pallas as pl
from jax.experimental.pallas import tpu as pltpu
from jax.experimental.pallas import tpu_sc as plsc

VOCAB = 100000
EMB = 128
OUT = 256
H = OUT // 2
T = 200
B = 1024
G4 = 4 * H

NC = 2
NS = 16
NW = NC * NS
CHUNK = 128
NSTEP = 8
SEG = 40


@functools.cache
def _make_sc_gather(rows_per_w, nchunk):
  mesh = plsc.VectorSubcoreMesh(core_axis_name="c", subcore_axis_name="s")

  @functools.partial(
      pl.kernel,
      mesh=mesh,
      out_type=jax.ShapeDtypeStruct((NW * rows_per_w, EMB), jnp.float32),
      scratch_types=[
          pltpu.VMEM((nchunk, CHUNK), jnp.int32),
          pltpu.VMEM((CHUNK, EMB), jnp.float32),
          pltpu.VMEM((CHUNK, EMB), jnp.float32),
          pltpu.SemaphoreType.DMA,
          pltpu.SemaphoreType.DMA,
          pltpu.SemaphoreType.DMA,
          pltpu.SemaphoreType.DMA,
      ],
  )
  def gather_kernel(table_hbm, idx_hbm, out_hbm, idx_v, rows0, rows1,
                    g0, g1, s0, s1):
    wid = lax.axis_index("s") * NC + lax.axis_index("c")
    base = wid * rows_per_w
    pltpu.sync_copy(idx_hbm.at[wid], idx_v)

    bufs = ((rows0, g0, s0), (rows1, g1, s1))

    def out_at(c):
      return out_hbm.at[pl.ds(base + c * CHUNK, CHUNK)]

    pltpu.async_copy(table_hbm.at[idx_v.at[0]], rows0, g0)
    pltpu.async_copy(table_hbm.at[idx_v.at[1]], rows1, g1)

    def body(i, carry):
      j = i * 2
      for k, (buf, gsem, ssem) in enumerate(bufs):
        c = j + k
        pltpu.make_async_copy(table_hbm.at[idx_v.at[c]], buf, gsem).wait()
        pltpu.async_copy(buf, out_at(c), ssem)

      for k, (buf, gsem, ssem) in enumerate(bufs):
        c = j + k + 2

        @pl.when(c < nchunk)
        def _():
          pltpu.make_async_copy(buf, out_at(c - 2), ssem).wait()
          pltpu.async_copy(table_hbm.at[idx_v.at[c]], buf, gsem)

      return carry

    lax.fori_loop(0, nchunk // 2, body, 0)
    pltpu.make_async_copy(rows0, out_at(nchunk - 2), s0).wait()
    pltpu.make_async_copy(rows1, out_at(nchunk - 1), s1).wait()

  return gather_kernel


def _gather(table, idx_flat):
  n = idx_flat.shape[0]
  rows_per_w = n // NW
  nchunk = rows_per_w // CHUNK
  idx3 = idx_flat.reshape(NW, nchunk, CHUNK)
  return _make_sc_gather(rows_per_w, nchunk)(table, idx3)


def _stage_kernel(xf_ref, xb_ref, wf_ref, bf_ref, wb_ref, bb_ref,
                  hf_in, cf_in, hb_in, cb_in,
                  hf_out, cf_out, hb_out, cb_out,
                  xhf, cf, xhb, cb):
  t = pl.program_id(0)
  nt = pl.num_programs(0)

  @pl.when(t == 0)
  def _init():
    xhf[:, EMB:] = hf_in[...].astype(jnp.bfloat16)
    cf[...] = cf_in[...]
    xhb[:, EMB:] = hb_in[...].astype(jnp.bfloat16)
    cb[...] = cb_in[...]

  def sig_pre(x):
    return 0.5 * jnp.tanh(x) + 0.5

  def step(x, xh, c_ref, w, b):
    xh[:, :EMB] = x.astype(jnp.bfloat16)
    gates = jnp.dot(xh[...], w,
                    preferred_element_type=jnp.float32).astype(jnp.bfloat16) + b
    i = sig_pre(gates[:, 0:H])
    f = sig_pre(gates[:, H:2 * H])
    g = jnp.tanh(gates[:, 2 * H:3 * H])
    o = sig_pre(gates[:, 3 * H:4 * H])
    c_new = f.astype(jnp.float32) * c_ref[...] + (i * g).astype(jnp.float32)
    h_new = o.astype(jnp.float32) * jnp.tanh(c_new)
    c_ref[...] = c_new
    xh[:, EMB:] = h_new.astype(jnp.bfloat16)
    return h_new

  for k in range(NSTEP):
    hf_new = step(xf_ref[k], xhf, cf, wf_ref[...], bf_ref[...])
    hb_new = step(xb_ref[NSTEP - 1 - k], xhb, cb, wb_ref[...], bb_ref[...])

  @pl.when(t == nt - 1)
  def _emit():
    hf_out[...] = hf_new
    cf_out[...] = cf[...]
    hb_out[...] = hb_new
    cb_out[...] = cb[...]


def _lstm_stage(xf_src, xb_src, f_map, b_map, nblocks,
                wf, bf, wb, bb, hf, cf, hb, cb):
  state_spec = pl.BlockSpec((B, H), lambda t: (0, 0))
  return pl.pallas_call(
      _stage_kernel,
      grid=(nblocks,),
      in_specs=[
          pl.BlockSpec((NSTEP, B, EMB), f_map),
          pl.BlockSpec((NSTEP, B, EMB), b_map),
          pl.BlockSpec((EMB + H, G4), lambda t: (0, 0)),
          pl.BlockSpec((1, G4), lambda t: (0, 0)),
          pl.BlockSpec((EMB + H, G4), lambda t: (0, 0)),
          pl.BlockSpec((1, G4), lambda t: (0, 0)),
          state_spec, state_spec, state_spec, state_spec,
      ],
      out_specs=[state_spec, state_spec, state_spec, state_spec],
      out_shape=[jax.ShapeDtypeStruct((B, H), jnp.float32)] * 4,
      scratch_shapes=[
          pltpu.VMEM((B, EMB + H), jnp.bfloat16),
          pltpu.VMEM((B, H), jnp.float32),
          pltpu.VMEM((B, EMB + H), jnp.bfloat16),
          pltpu.VMEM((B, H), jnp.float32),
      ],
      compiler_params=pltpu.CompilerParams(
          dimension_semantics=("arbitrary",),
      ),
  )(xf_src, xb_src, wf, bf, wb, bb, hf, cf, hb, cb)


def kernel(data, emb_table, Wf_ih, Wf_hh, bf_ih, bf_hh,
           Wb_ih, Wb_hh, bb_ih, bb_hh):
  data = data.astype(jnp.int32)
  idx_a = jnp.concatenate([data[:SEG], data[T - SEG:]], axis=0).reshape(-1)
  idx_b = data[SEG:T - SEG].reshape(-1)
  emb_a = _gather(emb_table, idx_a).reshape(2 * SEG, B, EMB)
  emb_b = _gather(emb_table, idx_b).reshape(T - 2 * SEG, B, EMB)

  gate_scale = jnp.concatenate([
      jnp.full((1, H), 0.5, jnp.float32),
      jnp.full((1, H), 0.5, jnp.float32),
      jnp.ones((1, H), jnp.float32),
      jnp.full((1, H), 0.5, jnp.float32),
  ], axis=1)
  wf = (jnp.concatenate([Wf_ih.T, Wf_hh.T], axis=0) * gate_scale).astype(jnp.bfloat16)
  bf = ((bf_ih + bf_hh).reshape(1, G4) * gate_scale).astype(jnp.bfloat16)
  wb = (jnp.concatenate([Wb_ih.T, Wb_hh.T], axis=0) * gate_scale).astype(jnp.bfloat16)
  bb = ((bb_ih + bb_hh).reshape(1, G4) * gate_scale).astype(jnp.bfloat16)

  z = jnp.zeros((B, H), jnp.float32)
  na = SEG // NSTEP
  nb = (T - 2 * SEG) // NSTEP
  st = _lstm_stage(emb_a, emb_a,
                   lambda t: (t, 0, 0), lambda t: (2 * na - 1 - t, 0, 0),
                   na, wf, bf, wb, bb, z, z, z, z)
  st = _lstm_stage(emb_b, emb_b,
                   lambda t: (t, 0, 0), lambda t: (nb - 1 - t, 0, 0),
                   nb, wf, bf, wb, bb, *st)
  hf, _, hb, _ = _lstm_stage(emb_a, emb_a,
                             lambda t: (na + t, 0, 0), lambda t: (na - 1 - t, 0, 0),
                             na, wf, bf, wb, bb, *st)
  return jnp.concatenate([hf, hb], axis=1)


# speedup vs baseline: 1.0246x; 1.0246x over previous
import functools

import jax
import jax.numpy as jnp
from jax import lax
from jax.experimental import pallas as pl
from jax.experimental.pallas import tpu as pltpu
from jax.experimental.pallas import tpu_sc as plsc

VOCAB = 100000
EMB = 128
OUT = 256
H = OUT // 2
T = 200
B = 1024
G4 = 4 * H

NC = 2
NS = 16
NW = NC * NS
CHUNK = 128
NSTEP = 8
SEG = 48


@functools.cache
def _make_sc_gather(rows_per_w, nchunk):
  mesh = plsc.VectorSubcoreMesh(core_axis_name="c", subcore_axis_name="s")

  @functools.partial(
      pl.kernel,
      mesh=mesh,
      out_type=jax.ShapeDtypeStruct((NW * rows_per_w, EMB), jnp.float32),
      scratch_types=[
          pltpu.VMEM((nchunk, CHUNK), jnp.int32),
          pltpu.VMEM((CHUNK, EMB), jnp.float32),
          pltpu.VMEM((CHUNK, EMB), jnp.float32),
          pltpu.SemaphoreType.DMA,
          pltpu.SemaphoreType.DMA,
          pltpu.SemaphoreType.DMA,
          pltpu.SemaphoreType.DMA,
      ],
  )
  def gather_kernel(table_hbm, idx_hbm, out_hbm, idx_v, rows0, rows1,
                    g0, g1, s0, s1):
    wid = lax.axis_index("s") * NC + lax.axis_index("c")
    base = wid * rows_per_w
    pltpu.sync_copy(idx_hbm.at[wid], idx_v)

    bufs = ((rows0, g0, s0), (rows1, g1, s1))

    def out_at(c):
      return out_hbm.at[pl.ds(base + c * CHUNK, CHUNK)]

    pltpu.async_copy(table_hbm.at[idx_v.at[0]], rows0, g0)
    pltpu.async_copy(table_hbm.at[idx_v.at[1]], rows1, g1)

    def body(i, carry):
      j = i * 2
      for k, (buf, gsem, ssem) in enumerate(bufs):
        c = j + k
        pltpu.make_async_copy(table_hbm.at[idx_v.at[c]], buf, gsem).wait()
        pltpu.async_copy(buf, out_at(c), ssem)

      for k, (buf, gsem, ssem) in enumerate(bufs):
        c = j + k + 2

        @pl.when(c < nchunk)
        def _():
          pltpu.make_async_copy(buf, out_at(c - 2), ssem).wait()
          pltpu.async_copy(table_hbm.at[idx_v.at[c]], buf, gsem)

      return carry

    lax.fori_loop(0, nchunk // 2, body, 0)
    pltpu.make_async_copy(rows0, out_at(nchunk - 2), s0).wait()
    pltpu.make_async_copy(rows1, out_at(nchunk - 1), s1).wait()

  return gather_kernel


def _gather(table, idx_flat):
  n = idx_flat.shape[0]
  rows_per_w = n // NW
  nchunk = rows_per_w // CHUNK
  idx3 = idx_flat.reshape(NW, nchunk, CHUNK)
  return _make_sc_gather(rows_per_w, nchunk)(table, idx3)


def _stage_kernel(xf_ref, xb_ref, wf_ref, bf_ref, wb_ref, bb_ref,
                  hf_in, cf_in, hb_in, cb_in,
                  hf_out, cf_out, hb_out, cb_out,
                  xhf, cf, xhb, cb):
  t = pl.program_id(0)
  nt = pl.num_programs(0)

  @pl.when(t == 0)
  def _init():
    xhf[:, EMB:] = hf_in[...].astype(jnp.bfloat16)
    cf[...] = cf_in[...]
    xhb[:, EMB:] = hb_in[...].astype(jnp.bfloat16)
    cb[...] = cb_in[...]

  def sig_pre(x):
    return 0.5 * jnp.tanh(x) + 0.5

  def step(x, xh, c_ref, w, b):
    xh[:, :EMB] = x.astype(jnp.bfloat16)
    gates = jnp.dot(xh[...], w,
                    preferred_element_type=jnp.float32).astype(jnp.bfloat16) + b
    i = sig_pre(gates[:, 0:H])
    f = sig_pre(gates[:, H:2 * H])
    g = jnp.tanh(gates[:, 2 * H:3 * H])
    o = sig_pre(gates[:, 3 * H:4 * H])
    c_new = f.astype(jnp.float32) * c_ref[...] + (i * g).astype(jnp.float32)
    h_new = o.astype(jnp.float32) * jnp.tanh(c_new)
    c_ref[...] = c_new
    xh[:, EMB:] = h_new.astype(jnp.bfloat16)
    return h_new

  for k in range(NSTEP):
    hf_new = step(xf_ref[k], xhf, cf, wf_ref[...], bf_ref[...])
    hb_new = step(xb_ref[NSTEP - 1 - k], xhb, cb, wb_ref[...], bb_ref[...])

  @pl.when(t == nt - 1)
  def _emit():
    hf_out[...] = hf_new
    cf_out[...] = cf[...]
    hb_out[...] = hb_new
    cb_out[...] = cb[...]


def _lstm_stage(xf_src, xb_src, f_map, b_map, nblocks,
                wf, bf, wb, bb, hf, cf, hb, cb):
  state_spec = pl.BlockSpec((B, H), lambda t: (0, 0))
  return pl.pallas_call(
      _stage_kernel,
      grid=(nblocks,),
      in_specs=[
          pl.BlockSpec((NSTEP, B, EMB), f_map),
          pl.BlockSpec((NSTEP, B, EMB), b_map),
          pl.BlockSpec((EMB + H, G4), lambda t: (0, 0)),
          pl.BlockSpec((1, G4), lambda t: (0, 0)),
          pl.BlockSpec((EMB + H, G4), lambda t: (0, 0)),
          pl.BlockSpec((1, G4), lambda t: (0, 0)),
          state_spec, state_spec, state_spec, state_spec,
      ],
      out_specs=[state_spec, state_spec, state_spec, state_spec],
      out_shape=[jax.ShapeDtypeStruct((B, H), jnp.float32)] * 4,
      scratch_shapes=[
          pltpu.VMEM((B, EMB + H), jnp.bfloat16),
          pltpu.VMEM((B, H), jnp.float32),
          pltpu.VMEM((B, EMB + H), jnp.bfloat16),
          pltpu.VMEM((B, H), jnp.float32),
      ],
      compiler_params=pltpu.CompilerParams(
          dimension_semantics=("arbitrary",),
      ),
  )(xf_src, xb_src, wf, bf, wb, bb, hf, cf, hb, cb)


def kernel(data, emb_table, Wf_ih, Wf_hh, bf_ih, bf_hh,
           Wb_ih, Wb_hh, bb_ih, bb_hh):
  data = data.astype(jnp.int32)
  idx_a = jnp.concatenate([data[:SEG], data[T - SEG:]], axis=0).reshape(-1)
  idx_b = data[SEG:T - SEG].reshape(-1)
  emb_a = _gather(emb_table, idx_a).reshape(2 * SEG, B, EMB)
  emb_b = _gather(emb_table, idx_b).reshape(T - 2 * SEG, B, EMB)

  gate_scale = jnp.concatenate([
      jnp.full((1, H), 0.5, jnp.float32),
      jnp.full((1, H), 0.5, jnp.float32),
      jnp.ones((1, H), jnp.float32),
      jnp.full((1, H), 0.5, jnp.float32),
  ], axis=1)
  wf = (jnp.concatenate([Wf_ih.T, Wf_hh.T], axis=0) * gate_scale).astype(jnp.bfloat16)
  bf = ((bf_ih + bf_hh).reshape(1, G4) * gate_scale).astype(jnp.bfloat16)
  wb = (jnp.concatenate([Wb_ih.T, Wb_hh.T], axis=0) * gate_scale).astype(jnp.bfloat16)
  bb = ((bb_ih + bb_hh).reshape(1, G4) * gate_scale).astype(jnp.bfloat16)

  z = jnp.zeros((B, H), jnp.float32)
  na = SEG // NSTEP
  nb = (T - 2 * SEG) // NSTEP
  st = _lstm_stage(emb_a, emb_a,
                   lambda t: (t, 0, 0), lambda t: (2 * na - 1 - t, 0, 0),
                   na, wf, bf, wb, bb, z, z, z, z)
  st = _lstm_stage(emb_b, emb_b,
                   lambda t: (t, 0, 0), lambda t: (nb - 1 - t, 0, 0),
                   nb, wf, bf, wb, bb, *st)
  hf, _, hb, _ = _lstm_stage(emb_a, emb_a,
                             lambda t: (na + t, 0, 0), lambda t: (na - 1 - t, 0, 0),
                             na, wf, bf, wb, bb, *st)
  return jnp.concatenate([hf, hb], axis=1)
